# Initial kernel scaffold; baseline (speedup 1.0000x reference)
import jax
import jax.numpy as jnp
from jax.experimental import pallas as pl


def kernel(x_cat_ids, tables):
    raise NotImplementedError("write your pallas kernel here")


# speedup vs baseline: 1.1497x; 1.1497x over previous
import functools

import jax
import jax.numpy as jnp
from jax import lax
from jax.experimental import pallas as pl
from jax.experimental.pallas import tpu as pltpu
from jax.experimental.pallas import tpu_sc as plsc

NUM_FIELDS = 26
VOCAB = 100000
EMB_DIM = 32
BATCH = 16384

NC = 2
NS = 16
NW = NC * NS

TOTAL_ROWS = BATCH * NUM_FIELDS
ROWS_PER_W = TOTAL_ROWS // NW
CHUNK = 128
NCHUNKS = ROWS_PER_W // CHUNK


def _build_kernel():
  mesh = plsc.VectorSubcoreMesh(
      core_axis_name="c", subcore_axis_name="s",
      num_cores=NC, num_subcores=NS)

  @functools.partial(
      pl.kernel,
      out_type=jax.ShapeDtypeStruct((NW, ROWS_PER_W, EMB_DIM), jnp.float32),
      mesh=mesh,
      scratch_types=[
          pltpu.VMEM((NCHUNKS, CHUNK), jnp.int32),
          pltpu.VMEM((CHUNK, EMB_DIM), jnp.float32),
          pltpu.SemaphoreType.DMA,
      ],
      compiler_params=pltpu.CompilerParams(use_tc_tiling_on_sc=False),
  )
  def emb_gather(idx_hbm, table_hbm, out_hbm, idx_v, rows, sem):
    wid = lax.axis_index("s") * NC + lax.axis_index("c")
    pltpu.sync_copy(idx_hbm.at[wid], idx_v)

    def step(c, carry):
      pltpu.async_copy(table_hbm.at[idx_v.at[c]], rows, sem).wait()
      pltpu.sync_copy(rows, out_hbm.at[wid, pl.ds(c * CHUNK, CHUNK)])
      return carry

    lax.fori_loop(0, NCHUNKS, step, 0)

  return emb_gather


_EMB_KERNEL = _build_kernel()


@jax.jit
def kernel(x_cat_ids, tables):
  offsets = (jnp.arange(NUM_FIELDS, dtype=jnp.int32) * VOCAB)[None, :]
  flat_idx = (x_cat_ids.astype(jnp.int32) + offsets).reshape(NW, NCHUNKS, CHUNK)
  flat_table = tables.reshape(NUM_FIELDS * VOCAB, EMB_DIM)
  out = _EMB_KERNEL(flat_idx, flat_table)
  return out.reshape(BATCH, NUM_FIELDS * EMB_DIM)


# speedup vs baseline: 1.2143x; 1.0561x over previous
import functools

import jax
import jax.numpy as jnp
from jax import lax
from jax.experimental import pallas as pl
from jax.experimental.pallas import tpu as pltpu
from jax.experimental.pallas import tpu_sc as plsc

NUM_FIELDS = 26
VOCAB = 100000
EMB_DIM = 32
BATCH = 16384

NC = 2
NS = 16
NW = NC * NS

TOTAL_ROWS = BATCH * NUM_FIELDS
ROWS_PER_W = TOTAL_ROWS // NW
CHUNK = 128
NCHUNKS = ROWS_PER_W // CHUNK
NBUF = 4


def _build_kernel():
  mesh = plsc.VectorSubcoreMesh(
      core_axis_name="c", subcore_axis_name="s",
      num_cores=NC, num_subcores=NS)

  @functools.partial(
      pl.kernel,
      out_type=jax.ShapeDtypeStruct((NW, ROWS_PER_W, EMB_DIM), jnp.float32),
      mesh=mesh,
      scratch_types=[
          pltpu.VMEM((NCHUNKS, CHUNK), jnp.int32),
          [pltpu.VMEM((CHUNK, EMB_DIM), jnp.float32) for _ in range(NBUF)],
          [pltpu.SemaphoreType.DMA for _ in range(NBUF)],
      ],
      compiler_params=pltpu.CompilerParams(use_tc_tiling_on_sc=False),
  )
  def emb_gather(idx_hbm, table_hbm, out_hbm, idx_v, rows, sems):
    wid = lax.axis_index("s") * NC + lax.axis_index("c")
    pltpu.sync_copy(idx_hbm.at[wid], idx_v)

    def gather(c, b):
      return pltpu.async_copy(table_hbm.at[idx_v.at[c]], rows[b], sems[b])

    def writeback(c, b):
      return pltpu.async_copy(
          rows[b], out_hbm.at[wid, pl.ds(c * CHUNK, CHUNK)], sems[b])

    for b in range(NBUF):
      gather(b, b)

    def step(i, carry):
      base = i * NBUF
      for b in range(NBUF):
        c = base + b
        pltpu.make_async_copy(table_hbm.at[idx_v.at[c]], rows[b],
                              sems[b]).wait()
        writeback(c, b).wait()
        gather(c + NBUF, b)
      return carry

    lax.fori_loop(0, NCHUNKS // NBUF - 1, step, 0)

    base = NCHUNKS - NBUF
    for b in range(NBUF):
      c = base + b
      pltpu.make_async_copy(table_hbm.at[idx_v.at[c]], rows[b], sems[b]).wait()
      writeback(c, b).wait()

  return emb_gather


_EMB_KERNEL = _build_kernel()


@jax.jit
def kernel(x_cat_ids, tables):
  offsets = (jnp.arange(NUM_FIELDS, dtype=jnp.int32) * VOCAB)[None, :]
  flat_idx = (x_cat_ids.astype(jnp.int32) + offsets).reshape(NW, NCHUNKS, CHUNK)
  flat_table = tables.reshape(NUM_FIELDS * VOCAB, EMB_DIM)
  out = _EMB_KERNEL(flat_idx, flat_table)
  return out.reshape(BATCH, NUM_FIELDS * EMB_DIM)
